# Initial kernel scaffold; baseline (speedup 1.0000x reference)
import jax
import jax.numpy as jnp
from jax.experimental import pallas as pl


def kernel(x, edge_index, Win, b_in, Wg0, bg0, Wg1, bg1, Wg2, bg2, Wd0, bd0, Wd1, bd1, Wd2, bd2, Wd3, bd3):
    raise NotImplementedError("write your pallas kernel here")


# speedup vs baseline: 3.8100x; 3.8100x over previous
import functools

import jax
import jax.numpy as jnp
from jax import lax
from jax.experimental import pallas as pl
from jax.experimental.pallas import tpu as pltpu
from jax.experimental.pallas import tpu_sc as plsc

NN = 10000
EE = 320000
F = 128
CHUNK = 64
NCHUNKS = EE // CHUNK
NC, NS = 2, 16
NW = NC * NS
CPW = NCHUNKS // NW
EXTRA = NCHUNKS - CPW * NW
RPT = 624
RREM = NN - RPT * NS
ZB = 48
NZB = RPT // ZB

_MESH = plsc.VectorSubcoreMesh(core_axis_name="c", subcore_axis_name="s")


def _edge_body(p_hbm, q_hbm, src_hbm, dst_hbm, s_out,
               acc, srcv, dstv, pbuf, qbuf, zbuf, sem1, sem2):
    c = lax.axis_index("c")
    s = lax.axis_index("s")
    wid = s * NC + c

    zero16 = jnp.zeros((16,), jnp.float32)

    def _zrow(r, _):
        for j in range(F // 16):
            zbuf[r, pl.ds(j * 16, 16)] = zero16
        return 0
    lax.fori_loop(0, ZB, _zrow, 0)
    for k in range(NZB):
        pltpu.sync_copy(zbuf, acc.at[pl.ds(s * RPT + k * ZB, ZB)])

    @pl.when(s == 0)
    def _():
        pltpu.sync_copy(zbuf.at[pl.ds(0, RREM)], acc.at[pl.ds(NS * RPT, RREM)])
    plsc.subcore_barrier()

    my_count = jnp.where(wid < EXTRA, CPW + 1, CPW)
    base = wid * CPW + jnp.minimum(wid, EXTRA)

    def _chunk(k, _):
        @pl.when(k < my_count)
        def _():
            e0 = (base + k) * CHUNK
            pltpu.sync_copy(src_hbm.at[pl.ds(e0, CHUNK)], srcv.at[0])
            pltpu.sync_copy(dst_hbm.at[pl.ds(e0, CHUNK)], dstv.at[0])
            cp_p = pltpu.async_copy(p_hbm.at[dstv.at[0]], pbuf, sem1)
            cp_q = pltpu.async_copy(q_hbm.at[srcv.at[0]], qbuf, sem2)
            cp_p.wait()
            cp_q.wait()

            def _row(r, _2):
                for j in range(F // 16):
                    sl = pl.ds(j * 16, 16)
                    z = pbuf[r, sl] + qbuf[r, sl]
                    pbuf[r, sl] = 1.0 / (1.0 + jnp.exp(z))
                return 0
            lax.fori_loop(0, CHUNK, _row, 0)
            pltpu.sync_copy(pbuf, acc.at[dstv.at[0]], add=True)
        return 0
    lax.fori_loop(0, CPW + 1, _chunk, 0)
    plsc.subcore_barrier()

    for k in range(NZB):
        r0 = s * RPT + k * ZB
        pltpu.sync_copy(acc.at[pl.ds(r0, ZB)], s_out.at[c, pl.ds(r0, ZB)])

    @pl.when(s == 0)
    def _():
        r0 = NS * RPT
        pltpu.sync_copy(acc.at[pl.ds(r0, RREM)], s_out.at[c, pl.ds(r0, RREM)])


_edge = pl.kernel(
    _edge_body,
    out_type=[jax.ShapeDtypeStruct((NC, NN, F), jnp.float32)],
    mesh=_MESH,
    scratch_types=[
        pltpu.VMEM_SHARED((NN, F), jnp.float32),
        pltpu.VMEM((1, CHUNK), jnp.int32),
        pltpu.VMEM((1, CHUNK), jnp.int32),
        pltpu.VMEM((CHUNK, F), jnp.float32),
        pltpu.VMEM((CHUNK, F), jnp.float32),
        pltpu.VMEM((ZB, F), jnp.float32),
        pltpu.SemaphoreType.DMA,
        pltpu.SemaphoreType.DMA,
    ],
)


def _count_body(dst_hbm, c_out, cacc, dstv, ones_b, zbuf, sem):
    c = lax.axis_index("c")
    s = lax.axis_index("s")
    wid = s * NC + c

    zero16 = jnp.zeros((16,), jnp.float32)
    one16 = jnp.ones((16,), jnp.float32)

    def _zrow(r, _):
        for j in range(F // 16):
            zbuf[r, pl.ds(j * 16, 16)] = zero16
        return 0
    lax.fori_loop(0, ZB, _zrow, 0)

    def _orow(r, _):
        for j in range(F // 16):
            ones_b[r, pl.ds(j * 16, 16)] = one16
        return 0
    lax.fori_loop(0, CHUNK, _orow, 0)
    for k in range(NZB):
        pltpu.sync_copy(zbuf, cacc.at[pl.ds(s * RPT + k * ZB, ZB)])

    @pl.when(s == 0)
    def _():
        pltpu.sync_copy(zbuf.at[pl.ds(0, RREM)], cacc.at[pl.ds(NS * RPT, RREM)])
    plsc.subcore_barrier()

    my_count = jnp.where(wid < EXTRA, CPW + 1, CPW)
    base = wid * CPW + jnp.minimum(wid, EXTRA)

    def _chunk(k, _):
        @pl.when(k < my_count)
        def _():
            e0 = (base + k) * CHUNK
            pltpu.sync_copy(dst_hbm.at[pl.ds(e0, CHUNK)], dstv.at[0])
            pltpu.sync_copy(ones_b, cacc.at[dstv.at[0]], add=True)
        return 0
    lax.fori_loop(0, CPW + 1, _chunk, 0)
    plsc.subcore_barrier()

    for k in range(NZB):
        r0 = s * RPT + k * ZB
        pltpu.sync_copy(cacc.at[pl.ds(r0, ZB)], c_out.at[c, pl.ds(r0, ZB)])

    @pl.when(s == 0)
    def _():
        r0 = NS * RPT
        pltpu.sync_copy(cacc.at[pl.ds(r0, RREM)], c_out.at[c, pl.ds(r0, RREM)])


_count = pl.kernel(
    _count_body,
    out_type=[jax.ShapeDtypeStruct((NC, NN, F), jnp.float32)],
    mesh=_MESH,
    scratch_types=[
        pltpu.VMEM_SHARED((NN, F), jnp.float32),
        pltpu.VMEM((1, CHUNK), jnp.int32),
        pltpu.VMEM((CHUNK, F), jnp.float32),
        pltpu.VMEM((ZB, F), jnp.float32),
        pltpu.SemaphoreType.DMA,
    ],
)


def _gather_body(h_hbm, src_hbm, dst_hbm, lout, rout,
                 srcv, dstv, hs, hd, sem1, sem2):
    c = lax.axis_index("c")
    s = lax.axis_index("s")
    wid = s * NC + c
    my_count = jnp.where(wid < EXTRA, CPW + 1, CPW)
    base = wid * CPW + jnp.minimum(wid, EXTRA)

    def _chunk(k, _):
        @pl.when(k < my_count)
        def _():
            e0 = (base + k) * CHUNK
            pltpu.sync_copy(src_hbm.at[pl.ds(e0, CHUNK)], srcv.at[0])
            pltpu.sync_copy(dst_hbm.at[pl.ds(e0, CHUNK)], dstv.at[0])
            cp_s = pltpu.async_copy(h_hbm.at[srcv.at[0]], hs, sem1)
            cp_d = pltpu.async_copy(h_hbm.at[dstv.at[0]], hd, sem2)
            cp_s.wait()
            cp_d.wait()
            pltpu.sync_copy(hs, lout.at[pl.ds(e0, CHUNK)])
            pltpu.sync_copy(hd, rout.at[pl.ds(e0, CHUNK)])
        return 0
    lax.fori_loop(0, CPW + 1, _chunk, 0)


_gather = pl.kernel(
    _gather_body,
    out_type=[jax.ShapeDtypeStruct((EE, F), jnp.float32),
              jax.ShapeDtypeStruct((EE, F), jnp.float32)],
    mesh=_MESH,
    scratch_types=[
        pltpu.VMEM((1, CHUNK), jnp.int32),
        pltpu.VMEM((1, CHUNK), jnp.int32),
        pltpu.VMEM((CHUNK, F), jnp.float32),
        pltpu.VMEM((CHUNK, F), jnp.float32),
        pltpu.SemaphoreType.DMA,
        pltpu.SemaphoreType.DMA,
    ],
)


_RB = 2000


def _pq0_body(x_ref, w_ref, b_ref, p_ref, q_ref):
    r = jnp.dot(x_ref[...], w_ref[...],
                preferred_element_type=jnp.float32,
                precision=lax.Precision.HIGHEST) + b_ref[...]
    p_ref[...] = r[:, :F]
    q_ref[...] = r[:, F:]


def _pq_body(s_ref, o_ref, a_ref, w_ref, b_ref, p_ref, q_ref):
    h = o_ref[...] - a_ref[...] * (s_ref[0] + s_ref[1])
    r = jnp.dot(h, w_ref[...], preferred_element_type=jnp.float32,
                precision=lax.Precision.HIGHEST) + b_ref[...]
    p_ref[...] = r[:, :F]
    q_ref[...] = r[:, F:]


def _dense_body(s_ref, o_ref, a_ref, w_ref, b_ref, out_ref):
    h = o_ref[...] - a_ref[...] * (s_ref[0] + s_ref[1])
    for i in range(4):
        h = jnp.tanh(jnp.dot(h, w_ref[i],
                             preferred_element_type=jnp.float32,
                precision=lax.Precision.HIGHEST) + b_ref[i])
    out_ref[...] = h


def _pq0(x, Wp, bp):
    return pl.pallas_call(
        _pq0_body,
        grid=(NN // _RB,),
        in_specs=[pl.BlockSpec((_RB, F), lambda i: (i, 0)),
                  pl.BlockSpec((F, 2 * F), lambda i: (0, 0)),
                  pl.BlockSpec((1, 2 * F), lambda i: (0, 0))],
        out_specs=[pl.BlockSpec((_RB, F), lambda i: (i, 0)),
                   pl.BlockSpec((_RB, F), lambda i: (i, 0))],
        out_shape=[jax.ShapeDtypeStruct((NN, F), jnp.float32)] * 2,
    )(x, Wp, bp)


def _pq(S, ones, alpha, Wp, bp):
    return pl.pallas_call(
        _pq_body,
        grid=(NN // _RB,),
        in_specs=[pl.BlockSpec((NC, _RB, F), lambda i: (0, i, 0)),
                  pl.BlockSpec((_RB, 1), lambda i: (i, 0)),
                  pl.BlockSpec((_RB, 1), lambda i: (i, 0)),
                  pl.BlockSpec((F, 2 * F), lambda i: (0, 0)),
                  pl.BlockSpec((1, 2 * F), lambda i: (0, 0))],
        out_specs=[pl.BlockSpec((_RB, F), lambda i: (i, 0)),
                   pl.BlockSpec((_RB, F), lambda i: (i, 0))],
        out_shape=[jax.ShapeDtypeStruct((NN, F), jnp.float32)] * 2,
    )(S, ones, alpha, Wp, bp)


def _dense(S, ones, alpha, Wd, bd):
    return pl.pallas_call(
        _dense_body,
        grid=(NN // _RB,),
        in_specs=[pl.BlockSpec((NC, _RB, F), lambda i: (0, i, 0)),
                  pl.BlockSpec((_RB, 1), lambda i: (i, 0)),
                  pl.BlockSpec((_RB, 1), lambda i: (i, 0)),
                  pl.BlockSpec((4, F, F), lambda i: (0, 0, 0)),
                  pl.BlockSpec((4, 1, F), lambda i: (0, 0, 0))],
        out_specs=pl.BlockSpec((_RB, F), lambda i: (i, 0)),
        out_shape=jax.ShapeDtypeStruct((NN, F), jnp.float32),
    )(S, ones, alpha, Wd, bd)


def _fold(W, b):
    Wt, Wb = W[:F], W[F:]
    Wp = jnp.concatenate([2.0 * (Wt - Wb), 2.0 * Wb], axis=1)
    bp = jnp.concatenate([2.0 * b, jnp.zeros_like(b)])[None, :]
    return Wp, bp


def kernel(x, edge_index, Win, b_in, Wg0, bg0, Wg1, bg1, Wg2, bg2,
           Wd0, bd0, Wd1, bd1, Wd2, bd2, Wd3, bd3):
    src = edge_index[0]
    dst = edge_index[1]

    Wp0, bp0 = _fold(Win, b_in)
    P, Q = _pq0(x, Wp0, bp0)
    (S,) = _edge(P, Q, src, dst)
    (C16,) = _count(dst)

    cnt = C16[0, :, 0:1] + C16[1, :, 0:1]
    ones = jnp.where(cnt > 0.0, 1.0, 0.0).astype(jnp.float32)
    alpha = 2.0 / jnp.maximum(cnt, 1.0)

    for (W, b) in ((Wg0, bg0), (Wg1, bg1), (Wg2, bg2)):
        Wp, bp = _fold(W, b)
        P, Q = _pq(S, ones, alpha, Wp, bp)
        (S,) = _edge(P, Q, src, dst)

    Wd = jnp.stack([Wd0, Wd1, Wd2, Wd3])
    bd = jnp.stack([bd0, bd1, bd2, bd3])[:, None, :]
    h = _dense(S, ones, alpha, Wd, bd)

    L, R = _gather(h, src, dst)
    x_cat = jnp.concatenate([L, R], axis=1)
    return (x_cat, edge_index)


# speedup vs baseline: 5.7834x; 1.5179x over previous
import functools

import jax
import jax.numpy as jnp
from jax import lax
from jax.experimental import pallas as pl
from jax.experimental.pallas import tpu as pltpu
from jax.experimental.pallas import tpu_sc as plsc

NN = 10000
EE = 320000
F = 128
CHUNK = 64
NCHUNKS = EE // CHUNK
NC, NS = 2, 16
NW = NC * NS
CPW = NCHUNKS // NW
EXTRA = NCHUNKS - CPW * NW
RPT = 624
RREM = NN - RPT * NS
ZB = 48
NZB = RPT // ZB

_MESH = plsc.VectorSubcoreMesh(core_axis_name="c", subcore_axis_name="s")


def _edge_body(p_hbm, q_hbm, src_hbm, dst_hbm, s_out,
               acc, idxv, pb0, qb0, pb1, qb1, sem0, sem1):
    c = lax.axis_index("c")
    s = lax.axis_index("s")
    wid = s * NC + c
    zero16 = jnp.zeros((16,), jnp.float32)

    def _zrow(r, _):
        for j in range(F // 16):
            pb0[r, pl.ds(j * 16, 16)] = zero16
        return 0
    lax.fori_loop(0, ZB, _zrow, 0)
    for k in range(NZB):
        pltpu.sync_copy(pb0.at[pl.ds(0, ZB)],
                        acc.at[pl.ds(s * RPT + k * ZB, ZB)])

    @pl.when(s == 0)
    def _():
        pltpu.sync_copy(pb0.at[pl.ds(0, RREM)], acc.at[pl.ds(NS * RPT, RREM)])
    plsc.subcore_barrier()

    my_count = jnp.where(wid < EXTRA, CPW + 1, CPW)
    base = wid * CPW + jnp.minimum(wid, EXTRA)
    bufs = ((pb0, qb0, sem0, 0), (pb1, qb1, sem1, 2))

    def _issue(k, b):
        pb, qb, sem, ir = bufs[b]
        e0 = (base + k) * CHUNK
        pltpu.sync_copy(src_hbm.at[pl.ds(e0, CHUNK)], idxv.at[ir])
        pltpu.sync_copy(dst_hbm.at[pl.ds(e0, CHUNK)], idxv.at[ir + 1])
        pltpu.async_copy(p_hbm.at[idxv.at[ir + 1]], pb, sem)
        pltpu.async_copy(q_hbm.at[idxv.at[ir]], qb, sem)

    def _finish(k, knext, b, bn):
        pb, qb, sem, ir = bufs[b]
        pltpu.make_async_copy(p_hbm.at[idxv.at[ir + 1]], pb, sem).wait()
        pltpu.make_async_copy(q_hbm.at[idxv.at[ir]], qb, sem).wait()

        @pl.when(knext < my_count)
        def _():
            _issue(knext, bn)

        def _row(r, _2):
            for j in range(F // 16):
                sl = pl.ds(j * 16, 16)
                z = pb[r, sl] + qb[r, sl]
                pb[r, sl] = 1.0 / (1.0 + jnp.exp(z))
            return 0
        lax.fori_loop(0, CHUNK, _row, 0)
        pltpu.sync_copy(pb, acc.at[idxv.at[ir + 1]], add=True)

    @pl.when(0 < my_count)
    def _():
        _issue(0, 0)

    def _g(g, _):
        k0 = 2 * g
        k1 = k0 + 1

        @pl.when(k0 < my_count)
        def _():
            _finish(k0, k1, 0, 1)

        @pl.when(k1 < my_count)
        def _():
            _finish(k1, k1 + 1, 1, 0)
        return 0
    lax.fori_loop(0, (CPW + 2) // 2, _g, 0)
    plsc.subcore_barrier()

    for k in range(NZB):
        r0 = s * RPT + k * ZB
        pltpu.sync_copy(acc.at[pl.ds(r0, ZB)], s_out.at[c, pl.ds(r0, ZB)])

    @pl.when(s == 0)
    def _():
        r0 = NS * RPT
        pltpu.sync_copy(acc.at[pl.ds(r0, RREM)], s_out.at[c, pl.ds(r0, RREM)])


_edge = pl.kernel(
    _edge_body,
    out_type=[jax.ShapeDtypeStruct((NC, NN, F), jnp.float32)],
    mesh=_MESH,
    scratch_types=[
        pltpu.VMEM_SHARED((NN, F), jnp.float32),
        pltpu.VMEM((4, CHUNK), jnp.int32),
        pltpu.VMEM((CHUNK, F), jnp.float32),
        pltpu.VMEM((CHUNK, F), jnp.float32),
        pltpu.VMEM((CHUNK, F), jnp.float32),
        pltpu.VMEM((CHUNK, F), jnp.float32),
        pltpu.SemaphoreType.DMA,
        pltpu.SemaphoreType.DMA,
    ],
)


def _count_body(dst_hbm, c_out, cacc, dstv, ones_b, zbuf, sem):
    c = lax.axis_index("c")
    s = lax.axis_index("s")
    wid = s * NC + c

    zero16 = jnp.zeros((16,), jnp.float32)
    one16 = jnp.ones((16,), jnp.float32)

    def _zrow(r, _):
        for j in range(F // 16):
            zbuf[r, pl.ds(j * 16, 16)] = zero16
        return 0
    lax.fori_loop(0, ZB, _zrow, 0)

    def _orow(r, _):
        for j in range(F // 16):
            ones_b[r, pl.ds(j * 16, 16)] = one16
        return 0
    lax.fori_loop(0, CHUNK, _orow, 0)
    for k in range(NZB):
        pltpu.sync_copy(zbuf, cacc.at[pl.ds(s * RPT + k * ZB, ZB)])

    @pl.when(s == 0)
    def _():
        pltpu.sync_copy(zbuf.at[pl.ds(0, RREM)], cacc.at[pl.ds(NS * RPT, RREM)])
    plsc.subcore_barrier()

    my_count = jnp.where(wid < EXTRA, CPW + 1, CPW)
    base = wid * CPW + jnp.minimum(wid, EXTRA)

    def _chunk(k, _):
        @pl.when(k < my_count)
        def _():
            e0 = (base + k) * CHUNK
            pltpu.sync_copy(dst_hbm.at[pl.ds(e0, CHUNK)], dstv.at[0])
            pltpu.sync_copy(ones_b, cacc.at[dstv.at[0]], add=True)
        return 0
    lax.fori_loop(0, CPW + 1, _chunk, 0)
    plsc.subcore_barrier()

    for k in range(NZB):
        r0 = s * RPT + k * ZB
        pltpu.sync_copy(cacc.at[pl.ds(r0, ZB)], c_out.at[c, pl.ds(r0, ZB)])

    @pl.when(s == 0)
    def _():
        r0 = NS * RPT
        pltpu.sync_copy(cacc.at[pl.ds(r0, RREM)], c_out.at[c, pl.ds(r0, RREM)])


_count = pl.kernel(
    _count_body,
    out_type=[jax.ShapeDtypeStruct((NC, NN, F), jnp.float32)],
    mesh=_MESH,
    scratch_types=[
        pltpu.VMEM_SHARED((NN, F), jnp.float32),
        pltpu.VMEM((1, CHUNK), jnp.int32),
        pltpu.VMEM((CHUNK, F), jnp.float32),
        pltpu.VMEM((ZB, F), jnp.float32),
        pltpu.SemaphoreType.DMA,
    ],
)


CG = 128
NCHG = EE // CG
CPWG = NCHG // NW
EXTRAG = NCHG - CPWG * NW


def _gather_body(h_hbm, src_hbm, dst_hbm, out_hbm,
                 idxv, hs0, hd0, hs1, hd1, sem0, sem1):
    c = lax.axis_index("c")
    s = lax.axis_index("s")
    wid = s * NC + c
    my_count = jnp.where(wid < EXTRAG, CPWG + 1, CPWG)
    base = wid * CPWG + jnp.minimum(wid, EXTRAG)
    bufs = ((hs0, hd0, sem0, 0), (hs1, hd1, sem1, 2))

    def _issue(k, b):
        hs, hd, sem, ir = bufs[b]
        e0 = (base + k) * CG
        pltpu.sync_copy(src_hbm.at[pl.ds(e0, CG)], idxv.at[ir])
        pltpu.sync_copy(dst_hbm.at[pl.ds(e0, CG)], idxv.at[ir + 1])
        pltpu.async_copy(h_hbm.at[idxv.at[ir]], hs, sem)
        pltpu.async_copy(h_hbm.at[idxv.at[ir + 1]], hd, sem)

    def _finish(k, knext, b, bn):
        hs, hd, sem, ir = bufs[b]
        pltpu.make_async_copy(h_hbm.at[idxv.at[ir]], hs, sem).wait()
        pltpu.make_async_copy(h_hbm.at[idxv.at[ir + 1]], hd, sem).wait()

        @pl.when(knext < my_count)
        def _():
            _issue(knext, bn)
        e0 = (base + k) * CG
        pltpu.sync_copy(hs, out_hbm.at[pl.ds(e0, CG), pl.ds(0, F)])
        pltpu.sync_copy(hd, out_hbm.at[pl.ds(e0, CG), pl.ds(F, F)])

    @pl.when(0 < my_count)
    def _():
        _issue(0, 0)

    def _g(g, _):
        k0 = 2 * g
        k1 = k0 + 1

        @pl.when(k0 < my_count)
        def _():
            _finish(k0, k1, 0, 1)

        @pl.when(k1 < my_count)
        def _():
            _finish(k1, k1 + 1, 1, 0)
        return 0
    lax.fori_loop(0, (CPWG + 2) // 2, _g, 0)


_gather = pl.kernel(
    _gather_body,
    out_type=jax.ShapeDtypeStruct((EE, 2 * F), jnp.float32),
    mesh=_MESH,
    scratch_types=[
        pltpu.VMEM((4, CG), jnp.int32),
        pltpu.VMEM((CG, F), jnp.float32),
        pltpu.VMEM((CG, F), jnp.float32),
        pltpu.VMEM((CG, F), jnp.float32),
        pltpu.VMEM((CG, F), jnp.float32),
        pltpu.SemaphoreType.DMA,
        pltpu.SemaphoreType.DMA,
    ],
)


_RB = 2000


def _pq0_body(x_ref, w_ref, b_ref, p_ref, q_ref):
    r = jnp.dot(x_ref[...], w_ref[...],
                preferred_element_type=jnp.float32,
                precision=lax.Precision.HIGHEST) + b_ref[...]
    p_ref[...] = r[:, :F]
    q_ref[...] = r[:, F:]


def _pq_body(s_ref, o_ref, a_ref, w_ref, b_ref, p_ref, q_ref):
    h = o_ref[...] - a_ref[...] * (s_ref[0] + s_ref[1])
    r = jnp.dot(h, w_ref[...], preferred_element_type=jnp.float32,
                precision=lax.Precision.HIGHEST) + b_ref[...]
    p_ref[...] = r[:, :F]
    q_ref[...] = r[:, F:]


def _dense_body(s_ref, o_ref, a_ref, w_ref, b_ref, out_ref):
    h = o_ref[...] - a_ref[...] * (s_ref[0] + s_ref[1])
    for i in range(4):
        h = jnp.tanh(jnp.dot(h, w_ref[i],
                             preferred_element_type=jnp.float32,
                precision=lax.Precision.HIGHEST) + b_ref[i])
    out_ref[...] = h


def _pq0(x, Wp, bp):
    return pl.pallas_call(
        _pq0_body,
        grid=(NN // _RB,),
        in_specs=[pl.BlockSpec((_RB, F), lambda i: (i, 0)),
                  pl.BlockSpec((F, 2 * F), lambda i: (0, 0)),
                  pl.BlockSpec((1, 2 * F), lambda i: (0, 0))],
        out_specs=[pl.BlockSpec((_RB, F), lambda i: (i, 0)),
                   pl.BlockSpec((_RB, F), lambda i: (i, 0))],
        out_shape=[jax.ShapeDtypeStruct((NN, F), jnp.float32)] * 2,
    )(x, Wp, bp)


def _pq(S, ones, alpha, Wp, bp):
    return pl.pallas_call(
        _pq_body,
        grid=(NN // _RB,),
        in_specs=[pl.BlockSpec((NC, _RB, F), lambda i: (0, i, 0)),
                  pl.BlockSpec((_RB, 1), lambda i: (i, 0)),
                  pl.BlockSpec((_RB, 1), lambda i: (i, 0)),
                  pl.BlockSpec((F, 2 * F), lambda i: (0, 0)),
                  pl.BlockSpec((1, 2 * F), lambda i: (0, 0))],
        out_specs=[pl.BlockSpec((_RB, F), lambda i: (i, 0)),
                   pl.BlockSpec((_RB, F), lambda i: (i, 0))],
        out_shape=[jax.ShapeDtypeStruct((NN, F), jnp.float32)] * 2,
    )(S, ones, alpha, Wp, bp)


def _dense(S, ones, alpha, Wd, bd):
    return pl.pallas_call(
        _dense_body,
        grid=(NN // _RB,),
        in_specs=[pl.BlockSpec((NC, _RB, F), lambda i: (0, i, 0)),
                  pl.BlockSpec((_RB, 1), lambda i: (i, 0)),
                  pl.BlockSpec((_RB, 1), lambda i: (i, 0)),
                  pl.BlockSpec((4, F, F), lambda i: (0, 0, 0)),
                  pl.BlockSpec((4, 1, F), lambda i: (0, 0, 0))],
        out_specs=pl.BlockSpec((_RB, F), lambda i: (i, 0)),
        out_shape=jax.ShapeDtypeStruct((NN, F), jnp.float32),
    )(S, ones, alpha, Wd, bd)


def _fold(W, b):
    Wt, Wb = W[:F], W[F:]
    Wp = jnp.concatenate([2.0 * (Wt - Wb), 2.0 * Wb], axis=1)
    bp = jnp.concatenate([2.0 * b, jnp.zeros_like(b)])[None, :]
    return Wp, bp


def kernel(x, edge_index, Win, b_in, Wg0, bg0, Wg1, bg1, Wg2, bg2,
           Wd0, bd0, Wd1, bd1, Wd2, bd2, Wd3, bd3):
    src = edge_index[0]
    dst = edge_index[1]

    (C16,) = _count(dst)
    Wp0, bp0 = _fold(Win, b_in)
    P, Q = _pq0(x, Wp0, bp0)
    (S,) = _edge(P, Q, src, dst)

    cnt = C16[0, :, 0:1] + C16[1, :, 0:1]
    ones = jnp.where(cnt > 0.0, 1.0, 0.0).astype(jnp.float32)
    alpha = 2.0 / jnp.maximum(cnt, 1.0)

    for (W, b) in ((Wg0, bg0), (Wg1, bg1), (Wg2, bg2)):
        Wp, bp = _fold(W, b)
        P, Q = _pq(S, ones, alpha, Wp, bp)
        (S,) = _edge(P, Q, src, dst)

    Wd = jnp.stack([Wd0, Wd1, Wd2, Wd3])
    bd = jnp.stack([bd0, bd1, bd2, bd3])[:, None, :]
    h = _dense(S, ones, alpha, Wd, bd)

    x_cat = _gather(h, src, dst)
    return (x_cat, edge_index)


# speedup vs baseline: 7.9626x; 1.3768x over previous
import functools

import jax
import jax.numpy as jnp
from jax import lax
from jax.experimental import pallas as pl
from jax.experimental.pallas import tpu as pltpu
from jax.experimental.pallas import tpu_sc as plsc

NN = 10000
EE = 320000
F = 128
CHUNK = 64
NCHUNKS = EE // CHUNK
NC, NS = 2, 16
NW = NC * NS
CPW = NCHUNKS // NW
EXTRA = NCHUNKS - CPW * NW
RPT = 624
RREM = NN - RPT * NS
ZB = 48
NZB = RPT // ZB

_MESH = plsc.VectorSubcoreMesh(core_axis_name="c", subcore_axis_name="s")


def _edge_body(p_hbm, q_hbm, src_hbm, dst_hbm, s_out,
               acc, idxv, pb0, qb0, pb1, qb1, sem0, sem1, semi0, semi1):
    c = lax.axis_index("c")
    s = lax.axis_index("s")
    wid = s * NC + c
    zero16 = jnp.zeros((16,), jnp.float32)

    def _zrow(r, _):
        for j in range(F // 16):
            pb0[r, pl.ds(j * 16, 16)] = zero16
        return 0
    lax.fori_loop(0, ZB, _zrow, 0)
    for k in range(NZB):
        pltpu.sync_copy(pb0.at[pl.ds(0, ZB)],
                        acc.at[pl.ds(s * RPT + k * ZB, ZB)])

    @pl.when(s == 0)
    def _():
        pltpu.sync_copy(pb0.at[pl.ds(0, RREM)], acc.at[pl.ds(NS * RPT, RREM)])
    plsc.subcore_barrier()

    my_count = jnp.where(wid < EXTRA, CPW + 1, CPW)
    base = wid * CPW + jnp.minimum(wid, EXTRA)
    bufs = ((pb0, qb0, sem0, semi0, 0), (pb1, qb1, sem1, semi1, 2))

    def _issue_idx(k, b):
        _, _, _, semi, ir = bufs[b]
        e0 = (base + k) * CHUNK
        pltpu.async_copy(src_hbm.at[pl.ds(e0, CHUNK)], idxv.at[ir], semi)
        pltpu.async_copy(dst_hbm.at[pl.ds(e0, CHUNK)], idxv.at[ir + 1], semi)

    def _wait_idx_issue_gather(k, b):
        pb, qb, sem, semi, ir = bufs[b]
        e0 = (base + k) * CHUNK
        pltpu.make_async_copy(src_hbm.at[pl.ds(e0, CHUNK)],
                              idxv.at[ir], semi).wait()
        pltpu.make_async_copy(dst_hbm.at[pl.ds(e0, CHUNK)],
                              idxv.at[ir + 1], semi).wait()
        pltpu.async_copy(p_hbm.at[idxv.at[ir + 1]], pb, sem)
        pltpu.async_copy(q_hbm.at[idxv.at[ir]], qb, sem)

    def _finish(k, b):
        pb, qb, sem, semi, ir = bufs[b]
        bn = 1 - b
        pltpu.make_async_copy(p_hbm.at[idxv.at[ir + 1]], pb, sem).wait()
        pltpu.make_async_copy(q_hbm.at[idxv.at[ir]], qb, sem).wait()
        def _sv(j, _):
            idxv[4 + b, pl.ds(j * 16, 16)] = idxv[ir + 1, pl.ds(j * 16, 16)]
            return 0
        lax.fori_loop(0, CHUNK // 16, _sv, 0)

        @pl.when(k + 2 < my_count)
        def _():
            _issue_idx(k + 2, b)

        @pl.when(k + 1 < my_count)
        def _():
            _wait_idx_issue_gather(k + 1, bn)

        def _row(r, _2):
            for j in range(F // 16):
                sl = pl.ds(j * 16, 16)
                z = pb[r, sl] + qb[r, sl]
                pb[r, sl] = 1.0 / (1.0 + jnp.exp(z))
            return 0
        lax.fori_loop(0, CHUNK, _row, 0)
        pltpu.sync_copy(pb, acc.at[idxv.at[4 + b]], add=True)

    _issue_idx(0, 0)
    _issue_idx(1, 1)
    _wait_idx_issue_gather(0, 0)

    def _g(g, _):
        k0 = 2 * g
        k1 = k0 + 1

        @pl.when(k0 < my_count)
        def _():
            _finish(k0, 0)

        @pl.when(k1 < my_count)
        def _():
            _finish(k1, 1)
        return 0
    lax.fori_loop(0, (CPW + 2) // 2, _g, 0)
    plsc.subcore_barrier()

    for k in range(NZB):
        r0 = s * RPT + k * ZB
        pltpu.sync_copy(acc.at[pl.ds(r0, ZB)], s_out.at[c, pl.ds(r0, ZB)])

    @pl.when(s == 0)
    def _():
        r0 = NS * RPT
        pltpu.sync_copy(acc.at[pl.ds(r0, RREM)], s_out.at[c, pl.ds(r0, RREM)])


_edge = pl.kernel(
    _edge_body,
    out_type=[jax.ShapeDtypeStruct((NC, NN, F), jnp.float32)],
    mesh=_MESH,
    scratch_types=[
        pltpu.VMEM_SHARED((NN, F), jnp.float32),
        pltpu.VMEM((6, CHUNK), jnp.int32),
        pltpu.VMEM((CHUNK, F), jnp.float32),
        pltpu.VMEM((CHUNK, F), jnp.float32),
        pltpu.VMEM((CHUNK, F), jnp.float32),
        pltpu.VMEM((CHUNK, F), jnp.float32),
        pltpu.SemaphoreType.DMA,
        pltpu.SemaphoreType.DMA,
        pltpu.SemaphoreType.DMA,
        pltpu.SemaphoreType.DMA,
    ],
)


def _count_body(dst_hbm, c_out, cacc, dstv, ones_b, zbuf, sem):
    c = lax.axis_index("c")
    s = lax.axis_index("s")
    wid = s * NC + c

    zero16 = jnp.zeros((16,), jnp.float32)
    one16 = jnp.ones((16,), jnp.float32)

    def _zrow(r, _):
        for j in range(F // 16):
            zbuf[r, pl.ds(j * 16, 16)] = zero16
        return 0
    lax.fori_loop(0, ZB, _zrow, 0)

    def _orow(r, _):
        for j in range(F // 16):
            ones_b[r, pl.ds(j * 16, 16)] = one16
        return 0
    lax.fori_loop(0, CHUNK, _orow, 0)
    for k in range(NZB):
        pltpu.sync_copy(zbuf, cacc.at[pl.ds(s * RPT + k * ZB, ZB)])

    @pl.when(s == 0)
    def _():
        pltpu.sync_copy(zbuf.at[pl.ds(0, RREM)], cacc.at[pl.ds(NS * RPT, RREM)])
    plsc.subcore_barrier()

    my_count = jnp.where(wid < EXTRA, CPW + 1, CPW)
    base = wid * CPW + jnp.minimum(wid, EXTRA)

    def _chunk(k, _):
        @pl.when(k < my_count)
        def _():
            e0 = (base + k) * CHUNK
            pltpu.sync_copy(dst_hbm.at[pl.ds(e0, CHUNK)], dstv.at[0])
            pltpu.sync_copy(ones_b, cacc.at[dstv.at[0]], add=True)
        return 0
    lax.fori_loop(0, CPW + 1, _chunk, 0)
    plsc.subcore_barrier()

    for k in range(NZB):
        r0 = s * RPT + k * ZB
        pltpu.sync_copy(cacc.at[pl.ds(r0, ZB)], c_out.at[c, pl.ds(r0, ZB)])

    @pl.when(s == 0)
    def _():
        r0 = NS * RPT
        pltpu.sync_copy(cacc.at[pl.ds(r0, RREM)], c_out.at[c, pl.ds(r0, RREM)])


_count = pl.kernel(
    _count_body,
    out_type=[jax.ShapeDtypeStruct((NC, NN, F), jnp.float32)],
    mesh=_MESH,
    scratch_types=[
        pltpu.VMEM_SHARED((NN, F), jnp.float32),
        pltpu.VMEM((1, CHUNK), jnp.int32),
        pltpu.VMEM((CHUNK, F), jnp.float32),
        pltpu.VMEM((ZB, F), jnp.float32),
        pltpu.SemaphoreType.DMA,
    ],
)


CG = 128
NCHG = EE // CG
CPWG = NCHG // NW
EXTRAG = NCHG - CPWG * NW


def _gather_body(h_hbm, src_hbm, dst_hbm, out_hbm,
                 idxv, hs0, hd0, hs1, hd1, sem0, sem1, semi0, semi1):
    c = lax.axis_index("c")
    s = lax.axis_index("s")
    wid = s * NC + c
    my_count = jnp.where(wid < EXTRAG, CPWG + 1, CPWG)
    base = wid * CPWG + jnp.minimum(wid, EXTRAG)
    bufs = ((hs0, hd0, sem0, semi0, 0), (hs1, hd1, sem1, semi1, 2))

    def _issue_idx(k, b):
        _, _, _, semi, ir = bufs[b]
        e0 = (base + k) * CG
        pltpu.async_copy(src_hbm.at[pl.ds(e0, CG)], idxv.at[ir], semi)
        pltpu.async_copy(dst_hbm.at[pl.ds(e0, CG)], idxv.at[ir + 1], semi)

    def _wait_idx_issue_gather(k, b):
        hs, hd, sem, semi, ir = bufs[b]
        e0 = (base + k) * CG
        pltpu.make_async_copy(src_hbm.at[pl.ds(e0, CG)],
                              idxv.at[ir], semi).wait()
        pltpu.make_async_copy(dst_hbm.at[pl.ds(e0, CG)],
                              idxv.at[ir + 1], semi).wait()
        pltpu.async_copy(h_hbm.at[idxv.at[ir]], hs, sem)
        pltpu.async_copy(h_hbm.at[idxv.at[ir + 1]], hd, sem)

    def _finish(k, b):
        hs, hd, sem, semi, ir = bufs[b]
        bn = 1 - b
        pltpu.make_async_copy(h_hbm.at[idxv.at[ir]], hs, sem).wait()
        pltpu.make_async_copy(h_hbm.at[idxv.at[ir + 1]], hd, sem).wait()

        @pl.when(k + 2 < my_count)
        def _():
            _issue_idx(k + 2, b)

        @pl.when(k + 1 < my_count)
        def _():
            _wait_idx_issue_gather(k + 1, bn)
        e0 = (base + k) * CG
        pltpu.sync_copy(hs, out_hbm.at[pl.ds(e0, CG), pl.ds(0, F)])
        pltpu.sync_copy(hd, out_hbm.at[pl.ds(e0, CG), pl.ds(F, F)])

    _issue_idx(0, 0)
    _issue_idx(1, 1)
    _wait_idx_issue_gather(0, 0)

    def _g(g, _):
        k0 = 2 * g
        k1 = k0 + 1

        @pl.when(k0 < my_count)
        def _():
            _finish(k0, 0)

        @pl.when(k1 < my_count)
        def _():
            _finish(k1, 1)
        return 0
    lax.fori_loop(0, (CPWG + 2) // 2, _g, 0)


_gather = pl.kernel(
    _gather_body,
    out_type=jax.ShapeDtypeStruct((EE, 2 * F), jnp.float32),
    mesh=_MESH,
    scratch_types=[
        pltpu.VMEM((4, CG), jnp.int32),
        pltpu.VMEM((CG, F), jnp.float32),
        pltpu.VMEM((CG, F), jnp.float32),
        pltpu.VMEM((CG, F), jnp.float32),
        pltpu.VMEM((CG, F), jnp.float32),
        pltpu.SemaphoreType.DMA,
        pltpu.SemaphoreType.DMA,
        pltpu.SemaphoreType.DMA,
        pltpu.SemaphoreType.DMA,
    ],
)


_RB = 2000


def _pq0_body(x_ref, w_ref, b_ref, p_ref, q_ref):
    r = jnp.dot(x_ref[...], w_ref[...],
                preferred_element_type=jnp.float32,
                precision=lax.Precision.HIGHEST) + b_ref[...]
    p_ref[...] = r[:, :F]
    q_ref[...] = r[:, F:]


def _pq_body(s_ref, o_ref, a_ref, w_ref, b_ref, p_ref, q_ref):
    h = o_ref[...] - a_ref[...] * (s_ref[0] + s_ref[1])
    r = jnp.dot(h, w_ref[...], preferred_element_type=jnp.float32,
                precision=lax.Precision.HIGHEST) + b_ref[...]
    p_ref[...] = r[:, :F]
    q_ref[...] = r[:, F:]


def _dense_body(s_ref, o_ref, a_ref, w_ref, b_ref, out_ref):
    h = o_ref[...] - a_ref[...] * (s_ref[0] + s_ref[1])
    for i in range(4):
        h = jnp.tanh(jnp.dot(h, w_ref[i],
                             preferred_element_type=jnp.float32,
                precision=lax.Precision.HIGHEST) + b_ref[i])
    out_ref[...] = h


def _pq0(x, Wp, bp):
    return pl.pallas_call(
        _pq0_body,
        grid=(NN // _RB,),
        in_specs=[pl.BlockSpec((_RB, F), lambda i: (i, 0)),
                  pl.BlockSpec((F, 2 * F), lambda i: (0, 0)),
                  pl.BlockSpec((1, 2 * F), lambda i: (0, 0))],
        out_specs=[pl.BlockSpec((_RB, F), lambda i: (i, 0)),
                   pl.BlockSpec((_RB, F), lambda i: (i, 0))],
        out_shape=[jax.ShapeDtypeStruct((NN, F), jnp.float32)] * 2,
    )(x, Wp, bp)


def _pq(S, ones, alpha, Wp, bp):
    return pl.pallas_call(
        _pq_body,
        grid=(NN // _RB,),
        in_specs=[pl.BlockSpec((NC, _RB, F), lambda i: (0, i, 0)),
                  pl.BlockSpec((_RB, 1), lambda i: (i, 0)),
                  pl.BlockSpec((_RB, 1), lambda i: (i, 0)),
                  pl.BlockSpec((F, 2 * F), lambda i: (0, 0)),
                  pl.BlockSpec((1, 2 * F), lambda i: (0, 0))],
        out_specs=[pl.BlockSpec((_RB, F), lambda i: (i, 0)),
                   pl.BlockSpec((_RB, F), lambda i: (i, 0))],
        out_shape=[jax.ShapeDtypeStruct((NN, F), jnp.float32)] * 2,
    )(S, ones, alpha, Wp, bp)


def _dense(S, ones, alpha, Wd, bd):
    return pl.pallas_call(
        _dense_body,
        grid=(NN // _RB,),
        in_specs=[pl.BlockSpec((NC, _RB, F), lambda i: (0, i, 0)),
                  pl.BlockSpec((_RB, 1), lambda i: (i, 0)),
                  pl.BlockSpec((_RB, 1), lambda i: (i, 0)),
                  pl.BlockSpec((4, F, F), lambda i: (0, 0, 0)),
                  pl.BlockSpec((4, 1, F), lambda i: (0, 0, 0))],
        out_specs=pl.BlockSpec((_RB, F), lambda i: (i, 0)),
        out_shape=jax.ShapeDtypeStruct((NN, F), jnp.float32),
    )(S, ones, alpha, Wd, bd)


def _fold(W, b):
    Wt, Wb = W[:F], W[F:]
    Wp = jnp.concatenate([2.0 * (Wt - Wb), 2.0 * Wb], axis=1)
    bp = jnp.concatenate([2.0 * b, jnp.zeros_like(b)])[None, :]
    return Wp, bp


def kernel(x, edge_index, Win, b_in, Wg0, bg0, Wg1, bg1, Wg2, bg2,
           Wd0, bd0, Wd1, bd1, Wd2, bd2, Wd3, bd3):
    src = edge_index[0]
    dst = edge_index[1]

    (C16,) = _count(dst)
    Wp0, bp0 = _fold(Win, b_in)
    P, Q = _pq0(x, Wp0, bp0)
    (S,) = _edge(P, Q, src, dst)

    cnt = C16[0, :, 0:1] + C16[1, :, 0:1]
    ones = jnp.where(cnt > 0.0, 1.0, 0.0).astype(jnp.float32)
    alpha = 2.0 / jnp.maximum(cnt, 1.0)

    for (W, b) in ((Wg0, bg0), (Wg1, bg1), (Wg2, bg2)):
        Wp, bp = _fold(W, b)
        P, Q = _pq(S, ones, alpha, Wp, bp)
        (S,) = _edge(P, Q, src, dst)

    Wd = jnp.stack([Wd0, Wd1, Wd2, Wd3])
    bd = jnp.stack([bd0, bd1, bd2, bd3])[:, None, :]
    h = _dense(S, ones, alpha, Wd, bd)

    x_cat = _gather(h, src, dst)
    return (x_cat, edge_index)


# speedup vs baseline: 8.4404x; 1.0600x over previous
import functools

import jax
import jax.numpy as jnp
from jax import lax
from jax.experimental import pallas as pl
from jax.experimental.pallas import tpu as pltpu
from jax.experimental.pallas import tpu_sc as plsc

NN = 10000
EE = 320000
F = 128
CHUNK = 80
NCHUNKS = EE // CHUNK
NC, NS = 2, 16
NW = NC * NS
CPW = NCHUNKS // NW
EXTRA = NCHUNKS - CPW * NW
RPT = 624
RREM = NN - RPT * NS
ZB = 48
NZB = RPT // ZB

_MESH = plsc.VectorSubcoreMesh(core_axis_name="c", subcore_axis_name="s")


def _edge_body(p_hbm, q_hbm, src_hbm, dst_hbm, s_out,
               acc, idxv, pb0, qb0, pb1, qb1, sem0, sem1, semi0, semi1):
    c = lax.axis_index("c")
    s = lax.axis_index("s")
    wid = s * NC + c
    zero16 = jnp.zeros((16,), jnp.float32)

    def _zrow(r, _):
        for j in range(F // 16):
            pb0[r, pl.ds(j * 16, 16)] = zero16
        return 0
    lax.fori_loop(0, ZB, _zrow, 0)
    for k in range(NZB):
        pltpu.sync_copy(pb0.at[pl.ds(0, ZB)],
                        acc.at[pl.ds(s * RPT + k * ZB, ZB)])

    @pl.when(s == 0)
    def _():
        pltpu.sync_copy(pb0.at[pl.ds(0, RREM)], acc.at[pl.ds(NS * RPT, RREM)])
    plsc.subcore_barrier()

    my_count = jnp.where(wid < EXTRA, CPW + 1, CPW)
    base = wid * CPW + jnp.minimum(wid, EXTRA)
    bufs = ((pb0, qb0, sem0, semi0, 0), (pb1, qb1, sem1, semi1, 2))

    def _issue_idx(k, b):
        _, _, _, semi, ir = bufs[b]
        e0 = (base + k) * CHUNK
        pltpu.async_copy(src_hbm.at[pl.ds(e0, CHUNK)], idxv.at[ir], semi)
        pltpu.async_copy(dst_hbm.at[pl.ds(e0, CHUNK)], idxv.at[ir + 1], semi)

    def _wait_idx_issue_gather(k, b):
        pb, qb, sem, semi, ir = bufs[b]
        e0 = (base + k) * CHUNK
        pltpu.make_async_copy(src_hbm.at[pl.ds(e0, CHUNK)],
                              idxv.at[ir], semi).wait()
        pltpu.make_async_copy(dst_hbm.at[pl.ds(e0, CHUNK)],
                              idxv.at[ir + 1], semi).wait()
        pltpu.async_copy(p_hbm.at[idxv.at[ir + 1]], pb, sem)
        pltpu.async_copy(q_hbm.at[idxv.at[ir]], qb, sem)

    def _finish(k, b):
        pb, qb, sem, semi, ir = bufs[b]
        bn = 1 - b
        pltpu.make_async_copy(p_hbm.at[idxv.at[ir + 1]], pb, sem).wait()
        pltpu.make_async_copy(q_hbm.at[idxv.at[ir]], qb, sem).wait()
        def _sv(j, _):
            idxv[4 + b, pl.ds(j * 16, 16)] = idxv[ir + 1, pl.ds(j * 16, 16)]
            return 0
        lax.fori_loop(0, CHUNK // 16, _sv, 0)

        @pl.when(k + 2 < my_count)
        def _():
            _issue_idx(k + 2, b)

        @pl.when(k + 1 < my_count)
        def _():
            _wait_idx_issue_gather(k + 1, bn)

        def _row(r, _2):
            for j in range(F // 16):
                sl = pl.ds(j * 16, 16)
                z = pb[r, sl] + qb[r, sl]
                pb[r, sl] = 1.0 / (1.0 + jnp.exp(z))
            return 0
        lax.fori_loop(0, CHUNK, _row, 0)
        pltpu.sync_copy(pb, acc.at[idxv.at[4 + b]], add=True)

    _issue_idx(0, 0)
    _issue_idx(1, 1)
    _wait_idx_issue_gather(0, 0)

    def _g(g, _):
        k0 = 2 * g
        k1 = k0 + 1

        @pl.when(k0 < my_count)
        def _():
            _finish(k0, 0)

        @pl.when(k1 < my_count)
        def _():
            _finish(k1, 1)
        return 0
    lax.fori_loop(0, (CPW + 2) // 2, _g, 0)
    plsc.subcore_barrier()

    for k in range(NZB):
        r0 = s * RPT + k * ZB
        pltpu.sync_copy(acc.at[pl.ds(r0, ZB)], s_out.at[c, pl.ds(r0, ZB)])

    @pl.when(s == 0)
    def _():
        r0 = NS * RPT
        pltpu.sync_copy(acc.at[pl.ds(r0, RREM)], s_out.at[c, pl.ds(r0, RREM)])


_edge = pl.kernel(
    _edge_body,
    out_type=[jax.ShapeDtypeStruct((NC, NN, F), jnp.float32)],
    mesh=_MESH,
    scratch_types=[
        pltpu.VMEM_SHARED((NN, F), jnp.float32),
        pltpu.VMEM((6, CHUNK), jnp.int32),
        pltpu.VMEM((CHUNK, F), jnp.float32),
        pltpu.VMEM((CHUNK, F), jnp.float32),
        pltpu.VMEM((CHUNK, F), jnp.float32),
        pltpu.VMEM((CHUNK, F), jnp.float32),
        pltpu.SemaphoreType.DMA,
        pltpu.SemaphoreType.DMA,
        pltpu.SemaphoreType.DMA,
        pltpu.SemaphoreType.DMA,
    ],
)


def _count_body(dst_hbm, c_out, cacc, idxv, ones_b, zbuf, semi0, semi1):
    c = lax.axis_index("c")
    s = lax.axis_index("s")
    wid = s * NC + c

    zero16 = jnp.zeros((16,), jnp.float32)
    one16 = jnp.ones((16,), jnp.float32)

    def _zrow(r, _):
        for j in range(F // 16):
            zbuf[r, pl.ds(j * 16, 16)] = zero16
        return 0
    lax.fori_loop(0, ZB, _zrow, 0)

    def _orow(r, _):
        for j in range(F // 16):
            ones_b[r, pl.ds(j * 16, 16)] = one16
        return 0
    lax.fori_loop(0, CHUNK, _orow, 0)
    for k in range(NZB):
        pltpu.sync_copy(zbuf, cacc.at[pl.ds(s * RPT + k * ZB, ZB)])

    @pl.when(s == 0)
    def _():
        pltpu.sync_copy(zbuf.at[pl.ds(0, RREM)], cacc.at[pl.ds(NS * RPT, RREM)])
    plsc.subcore_barrier()

    my_count = jnp.where(wid < EXTRA, CPW + 1, CPW)
    base = wid * CPW + jnp.minimum(wid, EXTRA)
    sems = (semi0, semi1)

    def _issue_idx(k, b):
        e0 = (base + k) * CHUNK
        pltpu.async_copy(dst_hbm.at[pl.ds(e0, CHUNK)], idxv.at[b], sems[b])

    def _finish(k, b):
        e0 = (base + k) * CHUNK
        pltpu.make_async_copy(dst_hbm.at[pl.ds(e0, CHUNK)],
                              idxv.at[b], sems[b]).wait()
        pltpu.sync_copy(ones_b, cacc.at[idxv.at[b]], add=True)

        @pl.when(k + 2 < my_count)
        def _():
            _issue_idx(k + 2, b)

    _issue_idx(0, 0)
    _issue_idx(1, 1)

    def _g(g, _):
        k0 = 2 * g
        k1 = k0 + 1

        @pl.when(k0 < my_count)
        def _():
            _finish(k0, 0)

        @pl.when(k1 < my_count)
        def _():
            _finish(k1, 1)
        return 0
    lax.fori_loop(0, (CPW + 2) // 2, _g, 0)
    plsc.subcore_barrier()

    for k in range(NZB):
        r0 = s * RPT + k * ZB
        pltpu.sync_copy(cacc.at[pl.ds(r0, ZB)], c_out.at[c, pl.ds(r0, ZB)])

    @pl.when(s == 0)
    def _():
        r0 = NS * RPT
        pltpu.sync_copy(cacc.at[pl.ds(r0, RREM)], c_out.at[c, pl.ds(r0, RREM)])


_count = pl.kernel(
    _count_body,
    out_type=[jax.ShapeDtypeStruct((NC, NN, F), jnp.float32)],
    mesh=_MESH,
    scratch_types=[
        pltpu.VMEM_SHARED((NN, F), jnp.float32),
        pltpu.VMEM((2, CHUNK), jnp.int32),
        pltpu.VMEM((CHUNK, F), jnp.float32),
        pltpu.VMEM((ZB, F), jnp.float32),
        pltpu.SemaphoreType.DMA,
        pltpu.SemaphoreType.DMA,
    ],
)


CG = 128
NCHG = EE // CG
CPWG = NCHG // NW
EXTRAG = NCHG - CPWG * NW


def _gather_body(h_hbm, src_hbm, dst_hbm, out_hbm,
                 idxv, hs0, hd0, hs1, hd1, sem0, sem1, semi0, semi1):
    c = lax.axis_index("c")
    s = lax.axis_index("s")
    wid = s * NC + c
    my_count = jnp.where(wid < EXTRAG, CPWG + 1, CPWG)
    base = wid * CPWG + jnp.minimum(wid, EXTRAG)
    bufs = ((hs0, hd0, sem0, semi0, 0), (hs1, hd1, sem1, semi1, 2))

    def _issue_idx(k, b):
        _, _, _, semi, ir = bufs[b]
        e0 = (base + k) * CG
        pltpu.async_copy(src_hbm.at[pl.ds(e0, CG)], idxv.at[ir], semi)
        pltpu.async_copy(dst_hbm.at[pl.ds(e0, CG)], idxv.at[ir + 1], semi)

    def _wait_idx_issue_gather(k, b):
        hs, hd, sem, semi, ir = bufs[b]
        e0 = (base + k) * CG
        pltpu.make_async_copy(src_hbm.at[pl.ds(e0, CG)],
                              idxv.at[ir], semi).wait()
        pltpu.make_async_copy(dst_hbm.at[pl.ds(e0, CG)],
                              idxv.at[ir + 1], semi).wait()
        pltpu.async_copy(h_hbm.at[idxv.at[ir]], hs, sem)
        pltpu.async_copy(h_hbm.at[idxv.at[ir + 1]], hd, sem)

    def _finish(k, b):
        hs, hd, sem, semi, ir = bufs[b]
        bn = 1 - b
        pltpu.make_async_copy(h_hbm.at[idxv.at[ir]], hs, sem).wait()
        pltpu.make_async_copy(h_hbm.at[idxv.at[ir + 1]], hd, sem).wait()

        @pl.when(k + 2 < my_count)
        def _():
            _issue_idx(k + 2, b)

        @pl.when(k + 1 < my_count)
        def _():
            _wait_idx_issue_gather(k + 1, bn)
        e0 = (base + k) * CG
        pltpu.sync_copy(hs, out_hbm.at[pl.ds(e0, CG), pl.ds(0, F)])
        pltpu.sync_copy(hd, out_hbm.at[pl.ds(e0, CG), pl.ds(F, F)])

    _issue_idx(0, 0)
    _issue_idx(1, 1)
    _wait_idx_issue_gather(0, 0)

    def _g(g, _):
        k0 = 2 * g
        k1 = k0 + 1

        @pl.when(k0 < my_count)
        def _():
            _finish(k0, 0)

        @pl.when(k1 < my_count)
        def _():
            _finish(k1, 1)
        return 0
    lax.fori_loop(0, (CPWG + 2) // 2, _g, 0)


_gather = pl.kernel(
    _gather_body,
    out_type=jax.ShapeDtypeStruct((EE, 2 * F), jnp.float32),
    mesh=_MESH,
    scratch_types=[
        pltpu.VMEM((4, CG), jnp.int32),
        pltpu.VMEM((CG, F), jnp.float32),
        pltpu.VMEM((CG, F), jnp.float32),
        pltpu.VMEM((CG, F), jnp.float32),
        pltpu.VMEM((CG, F), jnp.float32),
        pltpu.SemaphoreType.DMA,
        pltpu.SemaphoreType.DMA,
        pltpu.SemaphoreType.DMA,
        pltpu.SemaphoreType.DMA,
    ],
)


_RB = 2000


def _pq0_body(x_ref, w_ref, b_ref, p_ref, q_ref):
    r = jnp.dot(x_ref[...], w_ref[...],
                preferred_element_type=jnp.float32,
                precision=lax.Precision.HIGHEST) + b_ref[...]
    p_ref[...] = r[:, :F]
    q_ref[...] = r[:, F:]


def _pq_body(s_ref, o_ref, a_ref, w_ref, b_ref, p_ref, q_ref):
    h = o_ref[...] - a_ref[...] * (s_ref[0] + s_ref[1])
    r = jnp.dot(h, w_ref[...], preferred_element_type=jnp.float32,
                precision=lax.Precision.HIGHEST) + b_ref[...]
    p_ref[...] = r[:, :F]
    q_ref[...] = r[:, F:]


def _dense_body(s_ref, o_ref, a_ref, w_ref, b_ref, out_ref):
    h = o_ref[...] - a_ref[...] * (s_ref[0] + s_ref[1])
    for i in range(4):
        h = jnp.tanh(jnp.dot(h, w_ref[i],
                             preferred_element_type=jnp.float32,
                precision=lax.Precision.HIGHEST) + b_ref[i])
    out_ref[...] = h


def _pq0(x, Wp, bp):
    return pl.pallas_call(
        _pq0_body,
        grid=(NN // _RB,),
        in_specs=[pl.BlockSpec((_RB, F), lambda i: (i, 0)),
                  pl.BlockSpec((F, 2 * F), lambda i: (0, 0)),
                  pl.BlockSpec((1, 2 * F), lambda i: (0, 0))],
        out_specs=[pl.BlockSpec((_RB, F), lambda i: (i, 0)),
                   pl.BlockSpec((_RB, F), lambda i: (i, 0))],
        out_shape=[jax.ShapeDtypeStruct((NN, F), jnp.float32)] * 2,
    )(x, Wp, bp)


def _pq(S, ones, alpha, Wp, bp):
    return pl.pallas_call(
        _pq_body,
        grid=(NN // _RB,),
        in_specs=[pl.BlockSpec((NC, _RB, F), lambda i: (0, i, 0)),
                  pl.BlockSpec((_RB, 1), lambda i: (i, 0)),
                  pl.BlockSpec((_RB, 1), lambda i: (i, 0)),
                  pl.BlockSpec((F, 2 * F), lambda i: (0, 0)),
                  pl.BlockSpec((1, 2 * F), lambda i: (0, 0))],
        out_specs=[pl.BlockSpec((_RB, F), lambda i: (i, 0)),
                   pl.BlockSpec((_RB, F), lambda i: (i, 0))],
        out_shape=[jax.ShapeDtypeStruct((NN, F), jnp.float32)] * 2,
    )(S, ones, alpha, Wp, bp)


def _dense(S, ones, alpha, Wd, bd):
    return pl.pallas_call(
        _dense_body,
        grid=(NN // _RB,),
        in_specs=[pl.BlockSpec((NC, _RB, F), lambda i: (0, i, 0)),
                  pl.BlockSpec((_RB, 1), lambda i: (i, 0)),
                  pl.BlockSpec((_RB, 1), lambda i: (i, 0)),
                  pl.BlockSpec((4, F, F), lambda i: (0, 0, 0)),
                  pl.BlockSpec((4, 1, F), lambda i: (0, 0, 0))],
        out_specs=pl.BlockSpec((_RB, F), lambda i: (i, 0)),
        out_shape=jax.ShapeDtypeStruct((NN, F), jnp.float32),
    )(S, ones, alpha, Wd, bd)


def _fold(W, b):
    Wt, Wb = W[:F], W[F:]
    Wp = jnp.concatenate([2.0 * (Wt - Wb), 2.0 * Wb], axis=1)
    bp = jnp.concatenate([2.0 * b, jnp.zeros_like(b)])[None, :]
    return Wp, bp


def kernel(x, edge_index, Win, b_in, Wg0, bg0, Wg1, bg1, Wg2, bg2,
           Wd0, bd0, Wd1, bd1, Wd2, bd2, Wd3, bd3):
    src = edge_index[0]
    dst = edge_index[1]

    (C16,) = _count(dst)
    Wp0, bp0 = _fold(Win, b_in)
    P, Q = _pq0(x, Wp0, bp0)
    (S,) = _edge(P, Q, src, dst)

    cnt = C16[0, :, 0:1] + C16[1, :, 0:1]
    ones = jnp.where(cnt > 0.0, 1.0, 0.0).astype(jnp.float32)
    alpha = 2.0 / jnp.maximum(cnt, 1.0)

    for (W, b) in ((Wg0, bg0), (Wg1, bg1), (Wg2, bg2)):
        Wp, bp = _fold(W, b)
        P, Q = _pq(S, ones, alpha, Wp, bp)
        (S,) = _edge(P, Q, src, dst)

    Wd = jnp.stack([Wd0, Wd1, Wd2, Wd3])
    bd = jnp.stack([bd0, bd1, bd2, bd3])[:, None, :]
    h = _dense(S, ones, alpha, Wd, bd)

    x_cat = _gather(h, src, dst)
    return (x_cat, edge_index)


# speedup vs baseline: 9.5038x; 1.1260x over previous
import functools

import jax
import jax.numpy as jnp
from jax import lax
from jax.experimental import pallas as pl
from jax.experimental.pallas import tpu as pltpu
from jax.experimental.pallas import tpu_sc as plsc

NN = 10000
EE = 320000
F = 128
CHUNK = 80
NCHUNKS = EE // CHUNK
NC, NS = 2, 16
NW = NC * NS
CPW = NCHUNKS // NW
EXTRA = NCHUNKS - CPW * NW
RPT = 624
RREM = NN - RPT * NS
ZB = 48
NZB = RPT // ZB

_MESH = plsc.VectorSubcoreMesh(core_axis_name="c", subcore_axis_name="s")


def _edge_body(p_hbm, q_hbm, src_hbm, dst_hbm, s_out,
               acc, idxv, pb0, qb0, pb1, qb1, sem0, sem1, semi0, semi1):
    c = lax.axis_index("c")
    s = lax.axis_index("s")
    wid = s * NC + c
    zero16 = jnp.zeros((16,), jnp.float32)

    def _zrow(r, _):
        for j in range(F // 16):
            pb0[r, pl.ds(j * 16, 16)] = zero16
        return 0
    lax.fori_loop(0, ZB, _zrow, 0)
    for k in range(NZB):
        pltpu.sync_copy(pb0.at[pl.ds(0, ZB)],
                        acc.at[pl.ds(s * RPT + k * ZB, ZB)])

    @pl.when(s == 0)
    def _():
        pltpu.sync_copy(pb0.at[pl.ds(0, RREM)], acc.at[pl.ds(NS * RPT, RREM)])
    plsc.subcore_barrier()

    my_count = jnp.where(wid < EXTRA, CPW + 1, CPW)
    base = wid * CPW + jnp.minimum(wid, EXTRA)
    bufs = ((pb0, qb0, sem0, semi0, 0), (pb1, qb1, sem1, semi1, 2))

    def _issue_idx(k, b):
        _, _, _, semi, ir = bufs[b]
        e0 = (base + k) * CHUNK
        pltpu.async_copy(src_hbm.at[pl.ds(e0, CHUNK)], idxv.at[ir], semi)
        pltpu.async_copy(dst_hbm.at[pl.ds(e0, CHUNK)], idxv.at[ir + 1], semi)

    def _wait_idx_issue_gather(k, b):
        pb, qb, sem, semi, ir = bufs[b]
        e0 = (base + k) * CHUNK
        pltpu.make_async_copy(src_hbm.at[pl.ds(e0, CHUNK)],
                              idxv.at[ir], semi).wait()
        pltpu.make_async_copy(dst_hbm.at[pl.ds(e0, CHUNK)],
                              idxv.at[ir + 1], semi).wait()
        pltpu.async_copy(p_hbm.at[idxv.at[ir + 1]], pb, sem)
        pltpu.async_copy(q_hbm.at[idxv.at[ir]], qb, sem)

    def _finish(k, b):
        pb, qb, sem, semi, ir = bufs[b]
        bn = 1 - b
        pltpu.make_async_copy(p_hbm.at[idxv.at[ir + 1]], pb, sem).wait()
        pltpu.make_async_copy(q_hbm.at[idxv.at[ir]], qb, sem).wait()
        def _sv(j, _):
            idxv[4 + b, pl.ds(j * 16, 16)] = idxv[ir + 1, pl.ds(j * 16, 16)]
            return 0
        lax.fori_loop(0, CHUNK // 16, _sv, 0)

        @pl.when(k + 2 < my_count)
        def _():
            _issue_idx(k + 2, b)

        @pl.when(k + 1 < my_count)
        def _():
            _wait_idx_issue_gather(k + 1, bn)

        def _row(r4, _2):
            r = r4 * 4
            for rr in range(4):
                for j in range(F // 16):
                    sl = pl.ds(j * 16, 16)
                    z = pb[r + rr, sl] + qb[r + rr, sl]
                    pb[r + rr, sl] = 1.0 / (1.0 + jnp.exp(z))
            return 0
        lax.fori_loop(0, CHUNK // 4, _row, 0)
        pltpu.sync_copy(pb, acc.at[idxv.at[4 + b]], add=True)

    _issue_idx(0, 0)
    _issue_idx(1, 1)
    _wait_idx_issue_gather(0, 0)

    def _g(g, _):
        k0 = 2 * g
        k1 = k0 + 1

        @pl.when(k0 < my_count)
        def _():
            _finish(k0, 0)

        @pl.when(k1 < my_count)
        def _():
            _finish(k1, 1)
        return 0
    lax.fori_loop(0, (CPW + 2) // 2, _g, 0)
    plsc.subcore_barrier()

    for k in range(NZB):
        r0 = s * RPT + k * ZB
        pltpu.sync_copy(acc.at[pl.ds(r0, ZB)], s_out.at[c, pl.ds(r0, ZB)])

    @pl.when(s == 0)
    def _():
        r0 = NS * RPT
        pltpu.sync_copy(acc.at[pl.ds(r0, RREM)], s_out.at[c, pl.ds(r0, RREM)])


_edge = pl.kernel(
    _edge_body,
    out_type=[jax.ShapeDtypeStruct((NC, NN, F), jnp.float32)],
    mesh=_MESH,
    scratch_types=[
        pltpu.VMEM_SHARED((NN, F), jnp.float32),
        pltpu.VMEM((6, CHUNK), jnp.int32),
        pltpu.VMEM((CHUNK, F), jnp.float32),
        pltpu.VMEM((CHUNK, F), jnp.float32),
        pltpu.VMEM((CHUNK, F), jnp.float32),
        pltpu.VMEM((CHUNK, F), jnp.float32),
        pltpu.SemaphoreType.DMA,
        pltpu.SemaphoreType.DMA,
        pltpu.SemaphoreType.DMA,
        pltpu.SemaphoreType.DMA,
    ],
)


def _count_body(dst_hbm, c_out, cacc, idxv, ones_b, zbuf, semi0, semi1):
    c = lax.axis_index("c")
    s = lax.axis_index("s")
    wid = s * NC + c

    zero16 = jnp.zeros((16,), jnp.float32)
    one16 = jnp.ones((16,), jnp.float32)

    def _zrow(r, _):
        for j in range(F // 16):
            zbuf[r, pl.ds(j * 16, 16)] = zero16
        return 0
    lax.fori_loop(0, ZB, _zrow, 0)

    def _orow(r, _):
        for j in range(F // 16):
            ones_b[r, pl.ds(j * 16, 16)] = one16
        return 0
    lax.fori_loop(0, CHUNK, _orow, 0)
    for k in range(NZB):
        pltpu.sync_copy(zbuf, cacc.at[pl.ds(s * RPT + k * ZB, ZB)])

    @pl.when(s == 0)
    def _():
        pltpu.sync_copy(zbuf.at[pl.ds(0, RREM)], cacc.at[pl.ds(NS * RPT, RREM)])
    plsc.subcore_barrier()

    my_count = jnp.where(wid < EXTRA, CPW + 1, CPW)
    base = wid * CPW + jnp.minimum(wid, EXTRA)
    sems = (semi0, semi1)

    def _issue_idx(k, b):
        e0 = (base + k) * CHUNK
        pltpu.async_copy(dst_hbm.at[pl.ds(e0, CHUNK)], idxv.at[b], sems[b])

    def _finish(k, b):
        e0 = (base + k) * CHUNK
        pltpu.make_async_copy(dst_hbm.at[pl.ds(e0, CHUNK)],
                              idxv.at[b], sems[b]).wait()
        pltpu.sync_copy(ones_b, cacc.at[idxv.at[b]], add=True)

        @pl.when(k + 2 < my_count)
        def _():
            _issue_idx(k + 2, b)

    _issue_idx(0, 0)
    _issue_idx(1, 1)

    def _g(g, _):
        k0 = 2 * g
        k1 = k0 + 1

        @pl.when(k0 < my_count)
        def _():
            _finish(k0, 0)

        @pl.when(k1 < my_count)
        def _():
            _finish(k1, 1)
        return 0
    lax.fori_loop(0, (CPW + 2) // 2, _g, 0)
    plsc.subcore_barrier()

    for k in range(NZB):
        r0 = s * RPT + k * ZB
        pltpu.sync_copy(cacc.at[pl.ds(r0, ZB)], c_out.at[c, pl.ds(r0, ZB)])

    @pl.when(s == 0)
    def _():
        r0 = NS * RPT
        pltpu.sync_copy(cacc.at[pl.ds(r0, RREM)], c_out.at[c, pl.ds(r0, RREM)])


_count = pl.kernel(
    _count_body,
    out_type=[jax.ShapeDtypeStruct((NC, NN, F), jnp.float32)],
    mesh=_MESH,
    scratch_types=[
        pltpu.VMEM_SHARED((NN, F), jnp.float32),
        pltpu.VMEM((2, CHUNK), jnp.int32),
        pltpu.VMEM((CHUNK, F), jnp.float32),
        pltpu.VMEM((ZB, F), jnp.float32),
        pltpu.SemaphoreType.DMA,
        pltpu.SemaphoreType.DMA,
    ],
)


CG = 128
NCHG = EE // CG
CPWG = NCHG // NW
EXTRAG = NCHG - CPWG * NW


def _gather_body(h_hbm, src_hbm, dst_hbm, out_hbm,
                 idxv, hs0, hd0, hs1, hd1, sem0, sem1, semi0, semi1):
    c = lax.axis_index("c")
    s = lax.axis_index("s")
    wid = s * NC + c
    my_count = jnp.where(wid < EXTRAG, CPWG + 1, CPWG)
    base = wid * CPWG + jnp.minimum(wid, EXTRAG)
    bufs = ((hs0, hd0, sem0, semi0, 0), (hs1, hd1, sem1, semi1, 2))

    def _issue_idx(k, b):
        _, _, _, semi, ir = bufs[b]
        e0 = (base + k) * CG
        pltpu.async_copy(src_hbm.at[pl.ds(e0, CG)], idxv.at[ir], semi)
        pltpu.async_copy(dst_hbm.at[pl.ds(e0, CG)], idxv.at[ir + 1], semi)

    def _wait_idx_issue_gather(k, b):
        hs, hd, sem, semi, ir = bufs[b]
        e0 = (base + k) * CG
        pltpu.make_async_copy(src_hbm.at[pl.ds(e0, CG)],
                              idxv.at[ir], semi).wait()
        pltpu.make_async_copy(dst_hbm.at[pl.ds(e0, CG)],
                              idxv.at[ir + 1], semi).wait()
        pltpu.async_copy(h_hbm.at[idxv.at[ir]], hs, sem)
        pltpu.async_copy(h_hbm.at[idxv.at[ir + 1]], hd, sem)

    def _finish(k, b):
        hs, hd, sem, semi, ir = bufs[b]
        bn = 1 - b
        pltpu.make_async_copy(h_hbm.at[idxv.at[ir]], hs, sem).wait()
        pltpu.make_async_copy(h_hbm.at[idxv.at[ir + 1]], hd, sem).wait()

        @pl.when(k + 2 < my_count)
        def _():
            _issue_idx(k + 2, b)

        @pl.when(k + 1 < my_count)
        def _():
            _wait_idx_issue_gather(k + 1, bn)
        e0 = (base + k) * CG
        pltpu.sync_copy(hs, out_hbm.at[pl.ds(e0, CG), pl.ds(0, F)])
        pltpu.sync_copy(hd, out_hbm.at[pl.ds(e0, CG), pl.ds(F, F)])

    _issue_idx(0, 0)
    _issue_idx(1, 1)
    _wait_idx_issue_gather(0, 0)

    def _g(g, _):
        k0 = 2 * g
        k1 = k0 + 1

        @pl.when(k0 < my_count)
        def _():
            _finish(k0, 0)

        @pl.when(k1 < my_count)
        def _():
            _finish(k1, 1)
        return 0
    lax.fori_loop(0, (CPWG + 2) // 2, _g, 0)


_gather = pl.kernel(
    _gather_body,
    out_type=jax.ShapeDtypeStruct((EE, 2 * F), jnp.float32),
    mesh=_MESH,
    scratch_types=[
        pltpu.VMEM((4, CG), jnp.int32),
        pltpu.VMEM((CG, F), jnp.float32),
        pltpu.VMEM((CG, F), jnp.float32),
        pltpu.VMEM((CG, F), jnp.float32),
        pltpu.VMEM((CG, F), jnp.float32),
        pltpu.SemaphoreType.DMA,
        pltpu.SemaphoreType.DMA,
        pltpu.SemaphoreType.DMA,
        pltpu.SemaphoreType.DMA,
    ],
)


_RB = 2000


def _pq0_body(x_ref, w_ref, b_ref, p_ref, q_ref):
    r = jnp.dot(x_ref[...], w_ref[...],
                preferred_element_type=jnp.float32,
                precision=lax.Precision.HIGHEST) + b_ref[...]
    p_ref[...] = r[:, :F]
    q_ref[...] = r[:, F:]


def _pq_body(s_ref, o_ref, a_ref, w_ref, b_ref, p_ref, q_ref):
    h = o_ref[...] - a_ref[...] * (s_ref[0] + s_ref[1])
    r = jnp.dot(h, w_ref[...], preferred_element_type=jnp.float32,
                precision=lax.Precision.HIGHEST) + b_ref[...]
    p_ref[...] = r[:, :F]
    q_ref[...] = r[:, F:]


def _dense_body(s_ref, o_ref, a_ref, w_ref, b_ref, out_ref):
    h = o_ref[...] - a_ref[...] * (s_ref[0] + s_ref[1])
    for i in range(4):
        h = jnp.tanh(jnp.dot(h, w_ref[i],
                             preferred_element_type=jnp.float32,
                precision=lax.Precision.HIGHEST) + b_ref[i])
    out_ref[...] = h


def _pq0(x, Wp, bp):
    return pl.pallas_call(
        _pq0_body,
        grid=(NN // _RB,),
        in_specs=[pl.BlockSpec((_RB, F), lambda i: (i, 0)),
                  pl.BlockSpec((F, 2 * F), lambda i: (0, 0)),
                  pl.BlockSpec((1, 2 * F), lambda i: (0, 0))],
        out_specs=[pl.BlockSpec((_RB, F), lambda i: (i, 0)),
                   pl.BlockSpec((_RB, F), lambda i: (i, 0))],
        out_shape=[jax.ShapeDtypeStruct((NN, F), jnp.float32)] * 2,
    )(x, Wp, bp)


def _pq(S, ones, alpha, Wp, bp):
    return pl.pallas_call(
        _pq_body,
        grid=(NN // _RB,),
        in_specs=[pl.BlockSpec((NC, _RB, F), lambda i: (0, i, 0)),
                  pl.BlockSpec((_RB, 1), lambda i: (i, 0)),
                  pl.BlockSpec((_RB, 1), lambda i: (i, 0)),
                  pl.BlockSpec((F, 2 * F), lambda i: (0, 0)),
                  pl.BlockSpec((1, 2 * F), lambda i: (0, 0))],
        out_specs=[pl.BlockSpec((_RB, F), lambda i: (i, 0)),
                   pl.BlockSpec((_RB, F), lambda i: (i, 0))],
        out_shape=[jax.ShapeDtypeStruct((NN, F), jnp.float32)] * 2,
    )(S, ones, alpha, Wp, bp)


def _dense(S, ones, alpha, Wd, bd):
    return pl.pallas_call(
        _dense_body,
        grid=(NN // _RB,),
        in_specs=[pl.BlockSpec((NC, _RB, F), lambda i: (0, i, 0)),
                  pl.BlockSpec((_RB, 1), lambda i: (i, 0)),
                  pl.BlockSpec((_RB, 1), lambda i: (i, 0)),
                  pl.BlockSpec((4, F, F), lambda i: (0, 0, 0)),
                  pl.BlockSpec((4, 1, F), lambda i: (0, 0, 0))],
        out_specs=pl.BlockSpec((_RB, F), lambda i: (i, 0)),
        out_shape=jax.ShapeDtypeStruct((NN, F), jnp.float32),
    )(S, ones, alpha, Wd, bd)


def _fold(W, b):
    Wt, Wb = W[:F], W[F:]
    Wp = jnp.concatenate([2.0 * (Wt - Wb), 2.0 * Wb], axis=1)
    bp = jnp.concatenate([2.0 * b, jnp.zeros_like(b)])[None, :]
    return Wp, bp


def kernel(x, edge_index, Win, b_in, Wg0, bg0, Wg1, bg1, Wg2, bg2,
           Wd0, bd0, Wd1, bd1, Wd2, bd2, Wd3, bd3):
    src = edge_index[0]
    dst = edge_index[1]

    (C16,) = _count(dst)
    Wp0, bp0 = _fold(Win, b_in)
    P, Q = _pq0(x, Wp0, bp0)
    (S,) = _edge(P, Q, src, dst)

    cnt = C16[0, :, 0:1] + C16[1, :, 0:1]
    ones = jnp.where(cnt > 0.0, 1.0, 0.0).astype(jnp.float32)
    alpha = 2.0 / jnp.maximum(cnt, 1.0)

    for (W, b) in ((Wg0, bg0), (Wg1, bg1), (Wg2, bg2)):
        Wp, bp = _fold(W, b)
        P, Q = _pq(S, ones, alpha, Wp, bp)
        (S,) = _edge(P, Q, src, dst)

    Wd = jnp.stack([Wd0, Wd1, Wd2, Wd3])
    bd = jnp.stack([bd0, bd1, bd2, bd3])[:, None, :]
    h = _dense(S, ones, alpha, Wd, bd)

    x_cat = _gather(h, src, dst)
    return (x_cat, edge_index)
